# baseline (speedup 1.0000x reference)
import jax
import jax.numpy as jnp
from jax import lax
from jax.experimental import pallas as pl
from jax.experimental.pallas import tpu as pltpu
from jax.experimental.pallas import tpu_sc as plsc

_B = 4096
_DIM = 64
_NW = 32
_BPW = _B // _NW
_L = 16
_TM = 512
_PW = 1024


def _tc_pack(inT_ref, out_ref):
    out_ref[:, 0:_DIM] = jnp.swapaxes(inT_ref[...], 0, 1)


def _sc_tables(user_h, item_h, utp_h, itp_h, gu_h, gi_h,
               idxu_v, idxi_v, ru_v, ri_v, gu_v, gi_v, s1, s2):
    wid = lax.axis_index("s") * 2 + lax.axis_index("c")
    base = wid * _BPW
    pltpu.sync_copy(user_h.at[pl.ds(base, _BPW)], idxu_v)
    pltpu.sync_copy(item_h.at[pl.ds(base, _BPW)], idxi_v)
    cu = pltpu.async_copy(utp_h.at[idxu_v], ru_v, s1)
    ci = pltpu.async_copy(itp_h.at[idxi_v], ri_v, s2)
    cu.wait()
    ci.wait()
    for rows_v, dst_v in ((ru_v, gu_v), (ri_v, gi_v)):

        @pl.loop(0, _BPW)
        def _extract(k, rows_v=rows_v, dst_v=dst_v):
            for j in range(_DIM // _L):
                dst_v[k, pl.ds(j * _L, _L)] = rows_v[k, pl.ds(j * _L, _L)]

    pltpu.sync_copy(gu_v, gu_h.at[pl.ds(base, _BPW)])
    pltpu.sync_copy(gi_v, gi_h.at[pl.ds(base, _BPW)])


def _sc_bias(item_h, ib2_h, beta_h, idxi_v, brow_idx_v, brows_v, beta_v, sem):
    wid = lax.axis_index("s") * 2 + lax.axis_index("c")
    base = wid * _BPW
    pltpu.sync_copy(item_h.at[pl.ds(base, _BPW)], idxi_v)
    for g in range(_BPW // _L):
        v = idxi_v[pl.ds(g * _L, _L)]
        brow_idx_v[pl.ds(g * _L, _L)] = lax.shift_right_logical(v, 4)
    pltpu.async_copy(ib2_h.at[brow_idx_v], brows_v, sem).wait()
    for g in range(_BPW // _L):
        rid = lax.iota(jnp.int32, _L) + g * _L
        lane = lax.bitwise_and(idxi_v[pl.ds(g * _L, _L)], 15)
        beta_v[pl.ds(g * _L, _L)] = plsc.load_gather(brows_v, [rid, lane])
    pltpu.sync_copy(beta_v, beta_h.at[pl.ds(base, _BPW)])


def _tc_xui(beta_ref, gu_ref, gi_ref, out_ref, s_ref):
    @pl.when(pl.program_id(0) == 0)
    def _():
        prod = gu_ref[...] * gi_ref[...]
        ones = jnp.ones((1, _DIM), jnp.float32)
        s_ref[...] = jax.lax.dot_general(
            ones, prod, (((1,), (1,)), ((), ())),
            precision=jax.lax.Precision.HIGHEST,
            preferred_element_type=jnp.float32)
    out_ref[...] = beta_ref[...] + s_ref[...]


def kernel(user, item, user_table, item_table, item_bias):
    n_rows = user_table.shape[0]
    mesh = plsc.VectorSubcoreMesh(core_axis_name="c", subcore_axis_name="s")

    n_blk = -(-n_rows // _PW)
    pack = pl.pallas_call(
        _tc_pack,
        grid=(n_blk,),
        in_specs=[pl.BlockSpec((_DIM, _PW), lambda j: (0, j))],
        out_specs=pl.BlockSpec((_PW, 2 * _DIM), lambda j: (j, 0)),
        out_shape=jax.ShapeDtypeStruct((n_blk * _PW, 2 * _DIM), jnp.float32),
    )
    utp = pack(user_table.T)
    itp = pack(item_table.T)

    tables = pl.kernel(
        _sc_tables,
        mesh=mesh,
        out_type=[
            jax.ShapeDtypeStruct((_B, _DIM), jnp.float32),
            jax.ShapeDtypeStruct((_B, _DIM), jnp.float32),
        ],
        scratch_types=[
            pltpu.VMEM((_BPW,), jnp.int32),
            pltpu.VMEM((_BPW,), jnp.int32),
            pltpu.VMEM((_BPW, 2 * _DIM), jnp.float32),
            pltpu.VMEM((_BPW, 2 * _DIM), jnp.float32),
            pltpu.VMEM((_BPW, _DIM), jnp.float32),
            pltpu.VMEM((_BPW, _DIM), jnp.float32),
            pltpu.SemaphoreType.DMA,
            pltpu.SemaphoreType.DMA,
        ],
    )
    gamma_u, gamma_i = tables(user, item, utp, itp)

    bias = pl.kernel(
        _sc_bias,
        mesh=mesh,
        out_type=jax.ShapeDtypeStruct((_B,), jnp.float32),
        scratch_types=[
            pltpu.VMEM((_BPW,), jnp.int32),
            pltpu.VMEM((_BPW,), jnp.int32),
            pltpu.VMEM((_BPW, _L), jnp.float32),
            pltpu.VMEM((_BPW,), jnp.float32),
            pltpu.SemaphoreType.DMA,
        ],
        compiler_params=pltpu.CompilerParams(use_tc_tiling_on_sc=False,
                                             needs_layout_passes=False),
    )
    beta_flat = bias(item, item_bias.reshape(item_bias.shape[0] // _L, _L))
    beta_i = beta_flat.reshape(_B, 1)

    xui = pl.pallas_call(
        _tc_xui,
        grid=(_B // _TM,),
        in_specs=[
            pl.BlockSpec((_TM, 1), lambda i: (i, 0)),
            pl.BlockSpec((_B, _DIM), lambda i: (0, 0)),
            pl.BlockSpec((_B, _DIM), lambda i: (0, 0)),
        ],
        out_specs=pl.BlockSpec((_TM, _B), lambda i: (i, 0)),
        out_shape=jax.ShapeDtypeStruct((_B, _B), jnp.float32),
        scratch_shapes=[pltpu.VMEM((1, _B), jnp.float32)],
    )(beta_i, gamma_u, gamma_i)

    return (xui, beta_i, gamma_u, gamma_i)


# speedup vs baseline: 1.7338x; 1.7338x over previous
import jax
import jax.numpy as jnp
from jax import lax
from jax.experimental import pallas as pl
from jax.experimental.pallas import tpu as pltpu
from jax.experimental.pallas import tpu_sc as plsc

_B = 4096
_DIM = 64
_NW = 32
_BPW = _B // _NW
_L = 16
_TM = 512
_PW = 8192


def _tc_pack(inT_ref, out_ref):
    out_ref[:, 0:_DIM] = jnp.swapaxes(inT_ref[...], 0, 1)


def _sc_tables(user_h, item_h, utp_h, itp_h, gu_h, gi_h,
               idxu_v, idxi_v, ru_v, ri_v, gu_v, gi_v, s1, s2):
    wid = lax.axis_index("s") * 2 + lax.axis_index("c")
    base = wid * _BPW
    pltpu.sync_copy(user_h.at[pl.ds(base, _BPW)], idxu_v)
    pltpu.sync_copy(item_h.at[pl.ds(base, _BPW)], idxi_v)
    cu = pltpu.async_copy(utp_h.at[idxu_v], ru_v, s1)
    ci = pltpu.async_copy(itp_h.at[idxi_v], ri_v, s2)
    cu.wait()
    ci.wait()
    for rows_v, dst_v in ((ru_v, gu_v), (ri_v, gi_v)):

        @pl.loop(0, _BPW)
        def _extract(k, rows_v=rows_v, dst_v=dst_v):
            for j in range(_DIM // _L):
                dst_v[k, pl.ds(j * _L, _L)] = rows_v[k, pl.ds(j * _L, _L)]

    pltpu.sync_copy(gu_v, gu_h.at[pl.ds(base, _BPW)])
    pltpu.sync_copy(gi_v, gi_h.at[pl.ds(base, _BPW)])


def _sc_bias(item_h, ib2_h, beta_h, idxi_v, brow_idx_v, brows_v, beta_v, sem):
    wid = lax.axis_index("s") * 2 + lax.axis_index("c")
    base = wid * _BPW
    pltpu.sync_copy(item_h.at[pl.ds(base, _BPW)], idxi_v)
    for g in range(_BPW // _L):
        v = idxi_v[pl.ds(g * _L, _L)]
        brow_idx_v[pl.ds(g * _L, _L)] = lax.shift_right_logical(v, 4)
    pltpu.async_copy(ib2_h.at[brow_idx_v], brows_v, sem).wait()
    for g in range(_BPW // _L):
        rid = lax.iota(jnp.int32, _L) + g * _L
        lane = lax.bitwise_and(idxi_v[pl.ds(g * _L, _L)], 15)
        beta_v[pl.ds(g * _L, _L)] = plsc.load_gather(brows_v, [rid, lane])
    pltpu.sync_copy(beta_v, beta_h.at[pl.ds(base, _BPW)])


def _tc_xui(beta_ref, gu_ref, gi_ref, out_ref, s_ref):
    @pl.when(pl.program_id(0) == 0)
    def _():
        prod = gu_ref[...] * gi_ref[...]
        ones = jnp.ones((1, _DIM), jnp.float32)
        s_ref[...] = jax.lax.dot_general(
            ones, prod, (((1,), (1,)), ((), ())),
            precision=jax.lax.Precision.HIGHEST,
            preferred_element_type=jnp.float32)
    out_ref[...] = beta_ref[...] + s_ref[...]


def kernel(user, item, user_table, item_table, item_bias):
    n_rows = user_table.shape[0]
    mesh = plsc.VectorSubcoreMesh(core_axis_name="c", subcore_axis_name="s")

    n_blk = -(-n_rows // _PW)
    pack = pl.pallas_call(
        _tc_pack,
        grid=(n_blk,),
        in_specs=[pl.BlockSpec((_DIM, _PW), lambda j: (0, j))],
        out_specs=pl.BlockSpec((_PW, 2 * _DIM), lambda j: (j, 0)),
        out_shape=jax.ShapeDtypeStruct((n_blk * _PW, 2 * _DIM), jnp.float32),
    )
    utp = pack(user_table.T)
    itp = pack(item_table.T)

    tables = pl.kernel(
        _sc_tables,
        mesh=mesh,
        out_type=[
            jax.ShapeDtypeStruct((_B, _DIM), jnp.float32),
            jax.ShapeDtypeStruct((_B, _DIM), jnp.float32),
        ],
        scratch_types=[
            pltpu.VMEM((_BPW,), jnp.int32),
            pltpu.VMEM((_BPW,), jnp.int32),
            pltpu.VMEM((_BPW, 2 * _DIM), jnp.float32),
            pltpu.VMEM((_BPW, 2 * _DIM), jnp.float32),
            pltpu.VMEM((_BPW, _DIM), jnp.float32),
            pltpu.VMEM((_BPW, _DIM), jnp.float32),
            pltpu.SemaphoreType.DMA,
            pltpu.SemaphoreType.DMA,
        ],
    )
    gamma_u, gamma_i = tables(user, item, utp, itp)

    bias = pl.kernel(
        _sc_bias,
        mesh=mesh,
        out_type=jax.ShapeDtypeStruct((_B,), jnp.float32),
        scratch_types=[
            pltpu.VMEM((_BPW,), jnp.int32),
            pltpu.VMEM((_BPW,), jnp.int32),
            pltpu.VMEM((_BPW, _L), jnp.float32),
            pltpu.VMEM((_BPW,), jnp.float32),
            pltpu.SemaphoreType.DMA,
        ],
        compiler_params=pltpu.CompilerParams(use_tc_tiling_on_sc=False,
                                             needs_layout_passes=False),
    )
    beta_flat = bias(item, item_bias.reshape(item_bias.shape[0] // _L, _L))
    beta_i = beta_flat.reshape(_B, 1)

    xui = pl.pallas_call(
        _tc_xui,
        grid=(_B // _TM,),
        in_specs=[
            pl.BlockSpec((_TM, 1), lambda i: (i, 0)),
            pl.BlockSpec((_B, _DIM), lambda i: (0, 0)),
            pl.BlockSpec((_B, _DIM), lambda i: (0, 0)),
        ],
        out_specs=pl.BlockSpec((_TM, _B), lambda i: (i, 0)),
        out_shape=jax.ShapeDtypeStruct((_B, _B), jnp.float32),
        scratch_shapes=[pltpu.VMEM((1, _B), jnp.float32)],
    )(beta_i, gamma_u, gamma_i)

    return (xui, beta_i, gamma_u, gamma_i)


# speedup vs baseline: 1.7719x; 1.0220x over previous
import jax
import jax.numpy as jnp
from jax import lax
from jax.experimental import pallas as pl
from jax.experimental.pallas import tpu as pltpu
from jax.experimental.pallas import tpu_sc as plsc

_B = 4096
_DIM = 64
_NW = 32
_BPW = _B // _NW
_L = 16
_TM = 1024
_PW = 8192


def _tc_pack(inT_ref, out_ref):
    out_ref[:, 0:_DIM] = jnp.swapaxes(inT_ref[...], 0, 1)


def _sc_table(idx_h, tp_h, g_h, idx_v, rows_v, g_v, sem):
    wid = lax.axis_index("s") * 2 + lax.axis_index("c")
    base = wid * _BPW
    pltpu.sync_copy(idx_h.at[pl.ds(base, _BPW)], idx_v)
    pltpu.async_copy(tp_h.at[idx_v], rows_v, sem).wait()

    @pl.loop(0, _BPW)
    def _extract(k):
        for j in range(_DIM // _L):
            g_v[k, pl.ds(j * _L, _L)] = rows_v[k, pl.ds(j * _L, _L)]

    pltpu.sync_copy(g_v, g_h.at[pl.ds(base, _BPW)])


def _sc_bias(item_h, ib2_h, beta_h, idxi_v, brow_idx_v, brows_v, beta_v, sem):
    wid = lax.axis_index("s") * 2 + lax.axis_index("c")
    base = wid * _BPW
    pltpu.sync_copy(item_h.at[pl.ds(base, _BPW)], idxi_v)
    for g in range(_BPW // _L):
        v = idxi_v[pl.ds(g * _L, _L)]
        brow_idx_v[pl.ds(g * _L, _L)] = lax.shift_right_logical(v, 4)
    pltpu.async_copy(ib2_h.at[brow_idx_v], brows_v, sem).wait()
    for g in range(_BPW // _L):
        rid = lax.iota(jnp.int32, _L) + g * _L
        lane = lax.bitwise_and(idxi_v[pl.ds(g * _L, _L)], 15)
        beta_v[pl.ds(g * _L, _L)] = plsc.load_gather(brows_v, [rid, lane])
    pltpu.sync_copy(beta_v, beta_h.at[pl.ds(base, _BPW)])


def _tc_xui(beta_ref, gu_ref, gi_ref, out_ref, s_ref):
    @pl.when(pl.program_id(0) == 0)
    def _():
        prod = gu_ref[...] * gi_ref[...]
        ones = jnp.ones((1, _DIM), jnp.float32)
        s_ref[...] = jax.lax.dot_general(
            ones, prod, (((1,), (1,)), ((), ())),
            precision=jax.lax.Precision.HIGHEST,
            preferred_element_type=jnp.float32)
    out_ref[...] = beta_ref[...] + s_ref[...]


def kernel(user, item, user_table, item_table, item_bias):
    n_rows = user_table.shape[0]
    mesh = plsc.VectorSubcoreMesh(core_axis_name="c", subcore_axis_name="s")

    n_blk = -(-n_rows // _PW)
    pack = pl.pallas_call(
        _tc_pack,
        grid=(n_blk,),
        in_specs=[pl.BlockSpec((_DIM, _PW), lambda j: (0, j))],
        out_specs=pl.BlockSpec((_PW, 2 * _DIM), lambda j: (j, 0)),
        out_shape=jax.ShapeDtypeStruct((n_blk * _PW, 2 * _DIM), jnp.float32),
    )

    gather = pl.kernel(
        _sc_table,
        mesh=mesh,
        out_type=jax.ShapeDtypeStruct((_B, _DIM), jnp.float32),
        scratch_types=[
            pltpu.VMEM((_BPW,), jnp.int32),
            pltpu.VMEM((_BPW, 2 * _DIM), jnp.float32),
            pltpu.VMEM((_BPW, _DIM), jnp.float32),
            pltpu.SemaphoreType.DMA,
        ],
    )

    utp = pack(user_table.T)
    gamma_u = gather(user, utp)
    itp = pack(item_table.T)
    gamma_i = gather(item, itp)

    bias = pl.kernel(
        _sc_bias,
        mesh=mesh,
        out_type=jax.ShapeDtypeStruct((_B,), jnp.float32),
        scratch_types=[
            pltpu.VMEM((_BPW,), jnp.int32),
            pltpu.VMEM((_BPW,), jnp.int32),
            pltpu.VMEM((_BPW, _L), jnp.float32),
            pltpu.VMEM((_BPW,), jnp.float32),
            pltpu.SemaphoreType.DMA,
        ],
        compiler_params=pltpu.CompilerParams(use_tc_tiling_on_sc=False,
                                             needs_layout_passes=False),
    )
    beta_flat = bias(item, item_bias.reshape(item_bias.shape[0] // _L, _L))
    beta_i = beta_flat.reshape(_B, 1)

    xui = pl.pallas_call(
        _tc_xui,
        grid=(_B // _TM,),
        in_specs=[
            pl.BlockSpec((_TM, 1), lambda i: (i, 0)),
            pl.BlockSpec((_B, _DIM), lambda i: (0, 0)),
            pl.BlockSpec((_B, _DIM), lambda i: (0, 0)),
        ],
        out_specs=pl.BlockSpec((_TM, _B), lambda i: (i, 0)),
        out_shape=jax.ShapeDtypeStruct((_B, _B), jnp.float32),
        scratch_shapes=[pltpu.VMEM((1, _B), jnp.float32)],
    )(beta_i, gamma_u, gamma_i)

    return (xui, beta_i, gamma_u, gamma_i)
